# baseline (speedup 1.0000x reference)
import functools

import jax
import jax.numpy as jnp
from jax import lax
from jax.experimental import pallas as pl
from jax.experimental.pallas import tpu as pltpu
from jax.experimental.pallas import tpu_sc as plsc

N = 10000
C = 128
NC = 2
NS = 16
NW = NC * NS
L = 16
NP = 10112
ROWS_PER_TILE = NP // NS
K = 80
NBUF = 3

_mesh = plsc.VectorSubcoreMesh(core_axis_name="c", subcore_axis_name="s")


def _pad_edges(n_edges):
    q = NW * K * 2
    return -(-n_edges // q) * q


def _make_spmm(n_edges_padded):
    ne = n_edges_padded // NW
    n_chunks = ne // K

    @functools.partial(
        pl.kernel,
        mesh=_mesh,
        out_type=jax.ShapeDtypeStruct((NC, NP, C), jnp.float32),
        scratch_types=(
            [pltpu.VMEM_SHARED((NP, C), jnp.float32)]
            + [pltpu.VMEM((K, C), jnp.float32)] * NBUF
            + [pltpu.VMEM((K,), jnp.int32)] * NBUF
            + [pltpu.VMEM((K,), jnp.float32)] * NBUF
            + [pltpu.VMEM((K,), jnp.int32)] * NBUF
            + [pltpu.VMEM((K,), jnp.int32)] * NBUF
            + [pltpu.SemaphoreType.DMA] * NBUF
            + [pltpu.SemaphoreType.DMA] * NBUF
            + [pltpu.SemaphoreType.DMA] * NBUF
        ),
    )
    def spmm(table, pk, vals, out, acc, *ring):
        g = ring[:NBUF]
        pkb = ring[NBUF:2 * NBUF]
        valb = ring[2 * NBUF:3 * NBUF]
        srcb = ring[3 * NBUF:4 * NBUF]
        dstb = ring[4 * NBUF:5 * NBUF]
        isem = ring[5 * NBUF:6 * NBUF]
        gsem = ring[6 * NBUF:7 * NBUF]
        ssem = ring[7 * NBUF:]
        cid = lax.axis_index("c")
        sid = lax.axis_index("s")
        wid = cid * NS + sid
        e0 = wid * ne

        def _issue_idx(t, b):
            pltpu.async_copy(pk.at[pl.ds(e0 + t * K, K)], pkb[b], isem[b])
            pltpu.async_copy(vals.at[pl.ds(e0 + t * K, K)], valb[b], isem[b])

        def _wait_idx(t, b):
            pltpu.make_async_copy(pk.at[pl.ds(e0 + t * K, K)], pkb[b], isem[b]).wait()
            pltpu.make_async_copy(vals.at[pl.ds(e0 + t * K, K)], valb[b], isem[b]).wait()

        def _unpack(b):
            for grp in range(K // L):
                w = pkb[b][pl.ds(grp * L, L)]
                srcb[b][pl.ds(grp * L, L)] = w & 0x3FFF
                dstb[b][pl.ds(grp * L, L)] = w >> 14

        def _gather(b):
            pltpu.async_copy(table.at[srcb[b]], g[b], gsem[b])

        def _wait_gather(b):
            pltpu.make_async_copy(table.at[srcb[b]], g[b], gsem[b]).wait()

        def _zrow(i, _):
            for j in range(C // L):
                g[0][i, pl.ds(j * L, L)] = jnp.zeros((L,), jnp.float32)
            return 0
        lax.fori_loop(0, K, _zrow, 0)

        base_row = sid * ROWS_PER_TILE

        def _zacc(r, _):
            pltpu.sync_copy(g[0], acc.at[pl.ds(base_row + r * K, K)])
            return 0
        lax.fori_loop(0, ROWS_PER_TILE // K, _zacc, 0)
        rem = ROWS_PER_TILE % K
        if rem:
            pltpu.sync_copy(
                g[0].at[pl.ds(0, rem)],
                acc.at[pl.ds(base_row + (ROWS_PER_TILE // K) * K, rem)])
        plsc.subcore_barrier()

        def _wait_scatter(b):
            pltpu.make_async_copy(g[b], acc.at[dstb[b]], ssem[b]).wait()

        _issue_idx(0, 0)
        _issue_idx(1, 1)
        _issue_idx(2, 2)
        _wait_idx(0, 0)
        _unpack(0)
        _gather(0)
        _wait_idx(1, 1)
        _unpack(1)
        _gather(1)

        def _iter(u, _):
            for b in range(NBUF):
                t = u * NBUF + b
                bp = (b + 2) % NBUF

                @pl.when(t >= 1)
                def _():
                    _wait_scatter(bp)

                @pl.when(t + 2 < n_chunks)
                def _():
                    _wait_idx(t + 2, bp)
                    _unpack(bp)
                    _gather(bp)

                _wait_gather(b)

                for t2 in range(K // L):
                    v16 = valb[b][pl.ds(t2 * L, L)]
                    for i in range(L):
                        vb = jnp.full((L,), v16[i], jnp.float32)
                        row = t2 * L + i
                        for j in range(C // L):
                            g[b][row, pl.ds(j * L, L)] = (
                                g[b][row, pl.ds(j * L, L)] * vb)

                pltpu.async_copy(g[b], acc.at[dstb[b]], ssem[b], add=True)

                @pl.when(t + 3 < n_chunks)
                def _():
                    _issue_idx(t + 3, b)
            return 0
        lax.fori_loop(0, n_chunks // NBUF, _iter, 0)

        _wait_scatter((n_chunks - 1) % NBUF)
        plsc.subcore_barrier()
        pltpu.sync_copy(
            acc.at[pl.ds(sid * ROWS_PER_TILE, ROWS_PER_TILE)],
            out.at[cid, pl.ds(sid * ROWS_PER_TILE, ROWS_PER_TILE)],
        )

    return spmm


_SLAB = 320
_TAIL = N - 31 * _SLAB


def _make_combine(with_bias_relu):
    def body(p, bias, out, a, b, biasv, sem):
        del sem
        cid = lax.axis_index("c")
        sid = lax.axis_index("s")
        wid = cid * NS + sid
        if with_bias_relu:
            pltpu.sync_copy(bias, biasv)

        def _do(r0, rows):
            pltpu.sync_copy(p.at[0, pl.ds(r0, rows)], a.at[pl.ds(0, rows)])
            pltpu.sync_copy(p.at[1, pl.ds(r0, rows)], b.at[pl.ds(0, rows)])

            def _row(i, _):
                for j in range(C // L):
                    x = a[i, pl.ds(j * L, L)] + b[i, pl.ds(j * L, L)]
                    if with_bias_relu:
                        x = jnp.maximum(x + biasv[0, pl.ds(j * L, L)], 0.0)
                    a[i, pl.ds(j * L, L)] = x
                return 0
            lax.fori_loop(0, rows, _row, 0)
            pltpu.sync_copy(a.at[pl.ds(0, rows)], out.at[pl.ds(r0, rows)])

        @pl.when(wid < 31)
        def _():
            _do(wid * _SLAB, _SLAB)

        @pl.when(wid == 31)
        def _():
            _do(31 * _SLAB, _TAIL)

    if with_bias_relu:
        fn = body
    else:
        def fn(p, out, a, b, biasv, sem):
            return body(p, None, out, a, b, biasv, sem)

    return pl.kernel(
        fn,
        mesh=_mesh,
        out_type=jax.ShapeDtypeStruct((N, C), jnp.float32),
        scratch_types=[
            pltpu.VMEM((_SLAB, C), jnp.float32),
            pltpu.VMEM((_SLAB, C), jnp.float32),
            pltpu.VMEM((1, C), jnp.float32),
            pltpu.SemaphoreType.DMA,
        ],
    )


_FSLAB = NP * C // NS
_FZB = 2048


def _make_feat_scatter(nnz_padded):
    ne = nnz_padded // NW
    n_chunks = ne // K

    @functools.partial(
        pl.kernel,
        mesh=_mesh,
        out_type=jax.ShapeDtypeStruct((NC * NP * C,), jnp.float32),
        scratch_types=(
            [
                pltpu.VMEM_SHARED((NP * C,), jnp.float32),
                pltpu.VMEM((_FZB,), jnp.float32),
            ]
            + [pltpu.VMEM((K,), jnp.int32)] * 2
            + [pltpu.VMEM((K,), jnp.float32)] * 2
            + [pltpu.SemaphoreType.DMA] * 2
            + [pltpu.SemaphoreType.DMA] * 2
        ),
    )
    def fscat(fidx, vals, out, acc, zb, *ring):
        ib = ring[0:2]
        vb = ring[2:4]
        isem = ring[4:6]
        ssem = ring[6:8]
        cid = lax.axis_index("c")
        sid = lax.axis_index("s")
        wid = cid * NS + sid
        e0 = wid * ne

        def _zrow(i, _):
            zb[pl.ds(i * L, L)] = jnp.zeros((L,), jnp.float32)
            return 0
        lax.fori_loop(0, _FZB // L, _zrow, 0)

        w0 = sid * _FSLAB

        def _zacc(r, _):
            pltpu.sync_copy(zb, acc.at[pl.ds(w0 + r * _FZB, _FZB)])
            return 0
        lax.fori_loop(0, _FSLAB // _FZB, _zacc, 0)
        rem = _FSLAB % _FZB
        if rem:
            pltpu.sync_copy(
                zb.at[pl.ds(0, rem)],
                acc.at[pl.ds(w0 + (_FSLAB // _FZB) * _FZB, rem)])
        plsc.subcore_barrier()

        def _issue_idx(t, b):
            pltpu.async_copy(fidx.at[pl.ds(e0 + t * K, K)], ib[b], isem[b])
            pltpu.async_copy(vals.at[pl.ds(e0 + t * K, K)], vb[b], isem[b])

        def _wait_idx(t, b):
            pltpu.make_async_copy(fidx.at[pl.ds(e0 + t * K, K)], ib[b], isem[b]).wait()
            pltpu.make_async_copy(vals.at[pl.ds(e0 + t * K, K)], vb[b], isem[b]).wait()

        _issue_idx(0, 0)
        _issue_idx(1, 1)

        def _iter(u, _):
            for b in range(2):
                t = u * 2 + b
                _wait_idx(t, b)
                pltpu.async_copy(vb[b], acc.at[ib[b]], ssem[b], add=True).wait()

                @pl.when(t + 2 < n_chunks)
                def _():
                    _issue_idx(t + 2, b)
            return 0
        lax.fori_loop(0, n_chunks // 2, _iter, 0)

        plsc.subcore_barrier()
        pltpu.sync_copy(
            acc.at[pl.ds(w0, _FSLAB)],
            out.at[pl.ds(cid * NP * C + w0, _FSLAB)],
        )

    return fscat


_BM = 400


def _tc_combine_body(p_ref, o_ref):
    o_ref[...] = p_ref[0] + p_ref[1]


_tc_combine = pl.pallas_call(
    _tc_combine_body,
    grid=(N // _BM,),
    in_specs=[pl.BlockSpec((NC, _BM, C), lambda i: (0, i, 0))],
    out_specs=pl.BlockSpec((_BM, C), lambda i: (i, 0)),
    out_shape=jax.ShapeDtypeStruct((N, C), jnp.float32),
)


def _tc_matmul_body(p_ref, w_ref, b_ref, o_ref):
    x = p_ref[0] + p_ref[1]
    y = jnp.dot(x, w_ref[...], preferred_element_type=jnp.float32)
    o_ref[...] = jnp.maximum(y + b_ref[...], 0.0)


_tc_matmul = pl.pallas_call(
    _tc_matmul_body,
    grid=(N // _BM,),
    in_specs=[
        pl.BlockSpec((NC, _BM, C), lambda i: (0, i, 0)),
        pl.BlockSpec((C, C), lambda i: (0, 0)),
        pl.BlockSpec((1, C), lambda i: (0, 0)),
    ],
    out_specs=pl.BlockSpec((_BM, C), lambda i: (i, 0)),
    out_shape=jax.ShapeDtypeStruct((N, C), jnp.float32),
)


_E_PAD = _pad_edges(320000)
_NNZ_PAD = _pad_edges(128000)
_feat_scatter = _make_feat_scatter(_NNZ_PAD)
_spmm_adj = _make_spmm(_E_PAD)
_combine_plain = _make_combine(False)


def kernel(adj_indices, adj_values, feat_rows, feat_cols, feat_values, weight, bias):
    nnz = feat_values.shape[0]
    fidx = feat_rows.astype(jnp.int32) * C + feat_cols.astype(jnp.int32)
    fidx = jnp.pad(fidx, (0, _NNZ_PAD - nnz))
    fv = jnp.pad(feat_values, (0, _NNZ_PAD - nnz))
    ap = (adj_indices[0].astype(jnp.int32) << 14) | adj_indices[1].astype(jnp.int32)
    ap = jnp.pad(ap, (0, _E_PAD - 320000))
    av = jnp.pad(adj_values, (0, _E_PAD - 320000))

    pf = _feat_scatter(fidx, fv).reshape(NC, NP, C)
    base = _tc_matmul(pf, weight, bias)
    for _ in range(2):
        p = _spmm_adj(base, ap, av)
        base = _tc_combine(p)
    return base


# speedup vs baseline: 1.0190x; 1.0190x over previous
import functools

import jax
import jax.numpy as jnp
from jax import lax
from jax.experimental import pallas as pl
from jax.experimental.pallas import tpu as pltpu
from jax.experimental.pallas import tpu_sc as plsc

N = 10000
C = 128
NC = 2
NS = 16
NW = NC * NS
L = 16
NP = 10112
ROWS_PER_TILE = NP // NS
K = 112
NBUF = 3

_mesh = plsc.VectorSubcoreMesh(core_axis_name="c", subcore_axis_name="s")


def _pad_edges(n_edges):
    q = NW * K * 2
    return -(-n_edges // q) * q


def _make_spmm(n_edges_padded):
    ne = n_edges_padded // NW
    n_chunks = ne // K

    @functools.partial(
        pl.kernel,
        mesh=_mesh,
        out_type=jax.ShapeDtypeStruct((NC, NP, C), jnp.float32),
        scratch_types=(
            [pltpu.VMEM_SHARED((NP, C), jnp.float32)]
            + [pltpu.VMEM((K, C), jnp.float32)] * NBUF
            + [pltpu.VMEM((K,), jnp.int32)] * NBUF
            + [pltpu.VMEM((K,), jnp.float32)] * NBUF
            + [pltpu.VMEM((K,), jnp.int32)] * NBUF
            + [pltpu.VMEM((K,), jnp.int32)] * NBUF
            + [pltpu.SemaphoreType.DMA] * NBUF
            + [pltpu.SemaphoreType.DMA] * NBUF
            + [pltpu.SemaphoreType.DMA] * NBUF
        ),
    )
    def spmm(table, pk, vals, out, acc, *ring):
        g = ring[:NBUF]
        pkb = ring[NBUF:2 * NBUF]
        valb = ring[2 * NBUF:3 * NBUF]
        srcb = ring[3 * NBUF:4 * NBUF]
        dstb = ring[4 * NBUF:5 * NBUF]
        isem = ring[5 * NBUF:6 * NBUF]
        gsem = ring[6 * NBUF:7 * NBUF]
        ssem = ring[7 * NBUF:]
        cid = lax.axis_index("c")
        sid = lax.axis_index("s")
        wid = cid * NS + sid
        e0 = wid * ne

        def _issue_idx(t, b):
            pltpu.async_copy(pk.at[pl.ds(e0 + t * K, K)], pkb[b], isem[b])
            pltpu.async_copy(vals.at[pl.ds(e0 + t * K, K)], valb[b], isem[b])

        def _wait_idx(t, b):
            pltpu.make_async_copy(pk.at[pl.ds(e0 + t * K, K)], pkb[b], isem[b]).wait()
            pltpu.make_async_copy(vals.at[pl.ds(e0 + t * K, K)], valb[b], isem[b]).wait()

        def _unpack(b):
            for grp in range(K // L):
                w = pkb[b][pl.ds(grp * L, L)]
                srcb[b][pl.ds(grp * L, L)] = w & 0x3FFF
                dstb[b][pl.ds(grp * L, L)] = w >> 14

        def _gather(b):
            pltpu.async_copy(table.at[srcb[b]], g[b], gsem[b])

        def _wait_gather(b):
            pltpu.make_async_copy(table.at[srcb[b]], g[b], gsem[b]).wait()

        def _zrow(i, _):
            for j in range(C // L):
                g[0][i, pl.ds(j * L, L)] = jnp.zeros((L,), jnp.float32)
            return 0
        lax.fori_loop(0, K, _zrow, 0)

        base_row = sid * ROWS_PER_TILE

        def _zacc(r, _):
            pltpu.sync_copy(g[0], acc.at[pl.ds(base_row + r * K, K)])
            return 0
        lax.fori_loop(0, ROWS_PER_TILE // K, _zacc, 0)
        rem = ROWS_PER_TILE % K
        if rem:
            pltpu.sync_copy(
                g[0].at[pl.ds(0, rem)],
                acc.at[pl.ds(base_row + (ROWS_PER_TILE // K) * K, rem)])
        plsc.subcore_barrier()

        def _wait_scatter(b):
            pltpu.make_async_copy(g[b], acc.at[dstb[b]], ssem[b]).wait()

        _issue_idx(0, 0)
        _issue_idx(1, 1)
        _issue_idx(2, 2)
        _wait_idx(0, 0)
        _unpack(0)
        _gather(0)
        _wait_idx(1, 1)
        _unpack(1)
        _gather(1)

        def _iter(u, _):
            for b in range(NBUF):
                t = u * NBUF + b
                bp = (b + 2) % NBUF

                @pl.when(t >= 1)
                def _():
                    _wait_scatter(bp)

                @pl.when(t + 2 < n_chunks)
                def _():
                    _wait_idx(t + 2, bp)
                    _unpack(bp)
                    _gather(bp)

                _wait_gather(b)

                for t2 in range(K // L):
                    v16 = valb[b][pl.ds(t2 * L, L)]
                    for i in range(L):
                        vb = jnp.full((L,), v16[i], jnp.float32)
                        row = t2 * L + i
                        for j in range(C // L):
                            g[b][row, pl.ds(j * L, L)] = (
                                g[b][row, pl.ds(j * L, L)] * vb)

                pltpu.async_copy(g[b], acc.at[dstb[b]], ssem[b], add=True)

                @pl.when(t + 3 < n_chunks)
                def _():
                    _issue_idx(t + 3, b)
            return 0
        lax.fori_loop(0, n_chunks // NBUF, _iter, 0)

        _wait_scatter((n_chunks - 1) % NBUF)
        plsc.subcore_barrier()
        pltpu.sync_copy(
            acc.at[pl.ds(sid * ROWS_PER_TILE, ROWS_PER_TILE)],
            out.at[cid, pl.ds(sid * ROWS_PER_TILE, ROWS_PER_TILE)],
        )

    return spmm


_SLAB = 320
_TAIL = N - 31 * _SLAB


def _make_combine(with_bias_relu):
    def body(p, bias, out, a, b, biasv, sem):
        del sem
        cid = lax.axis_index("c")
        sid = lax.axis_index("s")
        wid = cid * NS + sid
        if with_bias_relu:
            pltpu.sync_copy(bias, biasv)

        def _do(r0, rows):
            pltpu.sync_copy(p.at[0, pl.ds(r0, rows)], a.at[pl.ds(0, rows)])
            pltpu.sync_copy(p.at[1, pl.ds(r0, rows)], b.at[pl.ds(0, rows)])

            def _row(i, _):
                for j in range(C // L):
                    x = a[i, pl.ds(j * L, L)] + b[i, pl.ds(j * L, L)]
                    if with_bias_relu:
                        x = jnp.maximum(x + biasv[0, pl.ds(j * L, L)], 0.0)
                    a[i, pl.ds(j * L, L)] = x
                return 0
            lax.fori_loop(0, rows, _row, 0)
            pltpu.sync_copy(a.at[pl.ds(0, rows)], out.at[pl.ds(r0, rows)])

        @pl.when(wid < 31)
        def _():
            _do(wid * _SLAB, _SLAB)

        @pl.when(wid == 31)
        def _():
            _do(31 * _SLAB, _TAIL)

    if with_bias_relu:
        fn = body
    else:
        def fn(p, out, a, b, biasv, sem):
            return body(p, None, out, a, b, biasv, sem)

    return pl.kernel(
        fn,
        mesh=_mesh,
        out_type=jax.ShapeDtypeStruct((N, C), jnp.float32),
        scratch_types=[
            pltpu.VMEM((_SLAB, C), jnp.float32),
            pltpu.VMEM((_SLAB, C), jnp.float32),
            pltpu.VMEM((1, C), jnp.float32),
            pltpu.SemaphoreType.DMA,
        ],
    )


_FSLAB = NP * C // NS
_FZB = 2048


def _make_feat_scatter(nnz_padded):
    ne = nnz_padded // NW
    n_chunks = ne // K

    @functools.partial(
        pl.kernel,
        mesh=_mesh,
        out_type=jax.ShapeDtypeStruct((NC * NP * C,), jnp.float32),
        scratch_types=(
            [
                pltpu.VMEM_SHARED((NP * C,), jnp.float32),
                pltpu.VMEM((_FZB,), jnp.float32),
            ]
            + [pltpu.VMEM((K,), jnp.int32)] * 2
            + [pltpu.VMEM((K,), jnp.float32)] * 2
            + [pltpu.SemaphoreType.DMA] * 2
            + [pltpu.SemaphoreType.DMA] * 2
        ),
    )
    def fscat(fidx, vals, out, acc, zb, *ring):
        ib = ring[0:2]
        vb = ring[2:4]
        isem = ring[4:6]
        ssem = ring[6:8]
        cid = lax.axis_index("c")
        sid = lax.axis_index("s")
        wid = cid * NS + sid
        e0 = wid * ne

        def _zrow(i, _):
            zb[pl.ds(i * L, L)] = jnp.zeros((L,), jnp.float32)
            return 0
        lax.fori_loop(0, _FZB // L, _zrow, 0)

        w0 = sid * _FSLAB

        def _zacc(r, _):
            pltpu.sync_copy(zb, acc.at[pl.ds(w0 + r * _FZB, _FZB)])
            return 0
        lax.fori_loop(0, _FSLAB // _FZB, _zacc, 0)
        rem = _FSLAB % _FZB
        if rem:
            pltpu.sync_copy(
                zb.at[pl.ds(0, rem)],
                acc.at[pl.ds(w0 + (_FSLAB // _FZB) * _FZB, rem)])
        plsc.subcore_barrier()

        def _issue_idx(t, b):
            pltpu.async_copy(fidx.at[pl.ds(e0 + t * K, K)], ib[b], isem[b])
            pltpu.async_copy(vals.at[pl.ds(e0 + t * K, K)], vb[b], isem[b])

        def _wait_idx(t, b):
            pltpu.make_async_copy(fidx.at[pl.ds(e0 + t * K, K)], ib[b], isem[b]).wait()
            pltpu.make_async_copy(vals.at[pl.ds(e0 + t * K, K)], vb[b], isem[b]).wait()

        _issue_idx(0, 0)
        _issue_idx(1, 1)

        def _iter(u, _):
            for b in range(2):
                t = u * 2 + b
                _wait_idx(t, b)
                pltpu.async_copy(vb[b], acc.at[ib[b]], ssem[b], add=True).wait()

                @pl.when(t + 2 < n_chunks)
                def _():
                    _issue_idx(t + 2, b)
            return 0
        lax.fori_loop(0, n_chunks // 2, _iter, 0)

        plsc.subcore_barrier()
        pltpu.sync_copy(
            acc.at[pl.ds(w0, _FSLAB)],
            out.at[pl.ds(cid * NP * C + w0, _FSLAB)],
        )

    return fscat


_BM = 400


def _tc_combine_body(p_ref, o_ref):
    o_ref[...] = p_ref[0] + p_ref[1]


_tc_combine = pl.pallas_call(
    _tc_combine_body,
    grid=(N // _BM,),
    in_specs=[pl.BlockSpec((NC, _BM, C), lambda i: (0, i, 0))],
    out_specs=pl.BlockSpec((_BM, C), lambda i: (i, 0)),
    out_shape=jax.ShapeDtypeStruct((N, C), jnp.float32),
)


def _tc_matmul_body(p_ref, w_ref, b_ref, o_ref):
    x = p_ref[0] + p_ref[1]
    y = jnp.dot(x, w_ref[...], preferred_element_type=jnp.float32)
    o_ref[...] = jnp.maximum(y + b_ref[...], 0.0)


_tc_matmul = pl.pallas_call(
    _tc_matmul_body,
    grid=(N // _BM,),
    in_specs=[
        pl.BlockSpec((NC, _BM, C), lambda i: (0, i, 0)),
        pl.BlockSpec((C, C), lambda i: (0, 0)),
        pl.BlockSpec((1, C), lambda i: (0, 0)),
    ],
    out_specs=pl.BlockSpec((_BM, C), lambda i: (i, 0)),
    out_shape=jax.ShapeDtypeStruct((N, C), jnp.float32),
)


_E_PAD = _pad_edges(320000)
_NNZ_PAD = _pad_edges(128000)
_feat_scatter = _make_feat_scatter(_NNZ_PAD)
_spmm_adj = _make_spmm(_E_PAD)
_combine_plain = _make_combine(False)


def kernel(adj_indices, adj_values, feat_rows, feat_cols, feat_values, weight, bias):
    nnz = feat_values.shape[0]
    fidx = feat_rows.astype(jnp.int32) * C + feat_cols.astype(jnp.int32)
    fidx = jnp.pad(fidx, (0, _NNZ_PAD - nnz))
    fv = jnp.pad(feat_values, (0, _NNZ_PAD - nnz))
    ap = (adj_indices[0].astype(jnp.int32) << 14) | adj_indices[1].astype(jnp.int32)
    ap = jnp.pad(ap, (0, _E_PAD - 320000))
    av = jnp.pad(adj_values, (0, _E_PAD - 320000))

    pf = _feat_scatter(fidx, fv).reshape(NC, NP, C)
    base = _tc_matmul(pf, weight, bias)
    for _ in range(2):
        p = _spmm_adj(base, ap, av)
        base = _tc_combine(p)
    return base


# speedup vs baseline: 1.0198x; 1.0008x over previous
import functools

import jax
import jax.numpy as jnp
from jax import lax
from jax.experimental import pallas as pl
from jax.experimental.pallas import tpu as pltpu
from jax.experimental.pallas import tpu_sc as plsc

N = 10000
C = 128
NC = 2
NS = 16
NW = NC * NS
L = 16
NP = 10112
ROWS_PER_TILE = NP // NS
K = 112
NBUF = 3

_mesh = plsc.VectorSubcoreMesh(core_axis_name="c", subcore_axis_name="s")


def _pad_edges(n_edges):
    q = NW * K * 2
    return -(-n_edges // q) * q


def _make_spmm(n_edges_padded):
    ne = n_edges_padded // NW
    n_chunks = ne // K

    @functools.partial(
        pl.kernel,
        mesh=_mesh,
        out_type=jax.ShapeDtypeStruct((NC, NP, C), jnp.float32),
        scratch_types=(
            [pltpu.VMEM_SHARED((NP, C), jnp.float32)]
            + [pltpu.VMEM((K, C), jnp.float32)] * NBUF
            + [pltpu.VMEM((K,), jnp.int32)] * NBUF
            + [pltpu.VMEM((K,), jnp.float32)] * NBUF
            + [pltpu.VMEM((K,), jnp.int32)] * NBUF
            + [pltpu.VMEM((K,), jnp.int32)] * NBUF
            + [pltpu.SemaphoreType.DMA] * NBUF
            + [pltpu.SemaphoreType.DMA] * NBUF
            + [pltpu.SemaphoreType.DMA] * NBUF
        ),
    )
    def spmm(table, pk, vals, out, acc, *ring):
        g = ring[:NBUF]
        pkb = ring[NBUF:2 * NBUF]
        valb = ring[2 * NBUF:3 * NBUF]
        srcb = ring[3 * NBUF:4 * NBUF]
        dstb = ring[4 * NBUF:5 * NBUF]
        isem = ring[5 * NBUF:6 * NBUF]
        gsem = ring[6 * NBUF:7 * NBUF]
        ssem = ring[7 * NBUF:]
        cid = lax.axis_index("c")
        sid = lax.axis_index("s")
        wid = cid * NS + sid
        e0 = wid * ne

        def _issue_idx(t, b):
            pltpu.async_copy(pk.at[pl.ds(e0 + t * K, K)], pkb[b], isem[b])
            pltpu.async_copy(vals.at[pl.ds(e0 + t * K, K)], valb[b], isem[b])

        def _wait_idx(t, b):
            pltpu.make_async_copy(pk.at[pl.ds(e0 + t * K, K)], pkb[b], isem[b]).wait()
            pltpu.make_async_copy(vals.at[pl.ds(e0 + t * K, K)], valb[b], isem[b]).wait()

        def _unpack(b):
            for grp in range(K // L):
                w = pkb[b][pl.ds(grp * L, L)]
                srcb[b][pl.ds(grp * L, L)] = w & 0x3FFF
                dstb[b][pl.ds(grp * L, L)] = w >> 14

        def _gather(b):
            pltpu.async_copy(table.at[srcb[b]], g[b], gsem[b])

        def _wait_gather(b):
            pltpu.make_async_copy(table.at[srcb[b]], g[b], gsem[b]).wait()

        def _zrow(i, _):
            for j in range(C // L):
                g[0][i, pl.ds(j * L, L)] = jnp.zeros((L,), jnp.float32)
            return 0
        lax.fori_loop(0, K, _zrow, 0)

        base_row = sid * ROWS_PER_TILE

        def _zacc(r, _):
            pltpu.sync_copy(g[0], acc.at[pl.ds(base_row + r * K, K)])
            return 0
        lax.fori_loop(0, ROWS_PER_TILE // K, _zacc, 0)
        rem = ROWS_PER_TILE % K
        if rem:
            pltpu.sync_copy(
                g[0].at[pl.ds(0, rem)],
                acc.at[pl.ds(base_row + (ROWS_PER_TILE // K) * K, rem)])
        plsc.subcore_barrier()

        def _wait_scatter(b):
            pltpu.make_async_copy(g[b], acc.at[dstb[b]], ssem[b]).wait()

        _issue_idx(0, 0)
        _issue_idx(1, 1)
        _issue_idx(2, 2)
        _wait_idx(0, 0)
        _unpack(0)
        _gather(0)
        _wait_idx(1, 1)
        _unpack(1)
        _gather(1)

        def _iter(u, _):
            for b in range(NBUF):
                t = u * NBUF + b
                bp = (b + 2) % NBUF

                @pl.when(t >= 1)
                def _():
                    _wait_scatter(bp)

                @pl.when(t + 2 < n_chunks)
                def _():
                    _wait_idx(t + 2, bp)
                    _unpack(bp)
                    _gather(bp)

                _wait_gather(b)

                for t2 in range(K // L):
                    v16 = valb[b][pl.ds(t2 * L, L)]
                    for i in range(L):
                        vb = jnp.full((L,), v16[i], jnp.float32)
                        row = t2 * L + i
                        for j in range(C // L):
                            g[b][row, pl.ds(j * L, L)] = (
                                g[b][row, pl.ds(j * L, L)] * vb)

                pltpu.async_copy(g[b], acc.at[dstb[b]], ssem[b], add=True)

                @pl.when(t + 3 < n_chunks)
                def _():
                    _issue_idx(t + 3, b)
            return 0
        lax.fori_loop(0, n_chunks // NBUF, _iter, 0)

        _wait_scatter((n_chunks - 1) % NBUF)
        plsc.subcore_barrier()
        pltpu.sync_copy(
            acc.at[pl.ds(sid * ROWS_PER_TILE, ROWS_PER_TILE)],
            out.at[cid, pl.ds(sid * ROWS_PER_TILE, ROWS_PER_TILE)],
        )

    return spmm


_FSLAB = NP * C // NS
_FZB = 2048


def _make_feat_scatter(nnz_padded):
    ne = nnz_padded // NW
    n_chunks = ne // K

    @functools.partial(
        pl.kernel,
        mesh=_mesh,
        out_type=jax.ShapeDtypeStruct((NC * NP * C,), jnp.float32),
        scratch_types=(
            [
                pltpu.VMEM_SHARED((NP * C,), jnp.float32),
                pltpu.VMEM((_FZB,), jnp.float32),
            ]
            + [pltpu.VMEM((K,), jnp.int32)] * 2
            + [pltpu.VMEM((K,), jnp.float32)] * 2
            + [pltpu.SemaphoreType.DMA] * 2
            + [pltpu.SemaphoreType.DMA] * 2
        ),
    )
    def fscat(fidx, vals, out, acc, zb, *ring):
        ib = ring[0:2]
        vb = ring[2:4]
        isem = ring[4:6]
        ssem = ring[6:8]
        cid = lax.axis_index("c")
        sid = lax.axis_index("s")
        wid = cid * NS + sid
        e0 = wid * ne

        def _zrow(i, _):
            zb[pl.ds(i * L, L)] = jnp.zeros((L,), jnp.float32)
            return 0
        lax.fori_loop(0, _FZB // L, _zrow, 0)

        w0 = sid * _FSLAB

        def _zacc(r, _):
            pltpu.sync_copy(zb, acc.at[pl.ds(w0 + r * _FZB, _FZB)])
            return 0
        lax.fori_loop(0, _FSLAB // _FZB, _zacc, 0)
        rem = _FSLAB % _FZB
        if rem:
            pltpu.sync_copy(
                zb.at[pl.ds(0, rem)],
                acc.at[pl.ds(w0 + (_FSLAB // _FZB) * _FZB, rem)])
        plsc.subcore_barrier()

        def _issue_idx(t, b):
            pltpu.async_copy(fidx.at[pl.ds(e0 + t * K, K)], ib[b], isem[b])
            pltpu.async_copy(vals.at[pl.ds(e0 + t * K, K)], vb[b], isem[b])

        def _wait_idx(t, b):
            pltpu.make_async_copy(fidx.at[pl.ds(e0 + t * K, K)], ib[b], isem[b]).wait()
            pltpu.make_async_copy(vals.at[pl.ds(e0 + t * K, K)], vb[b], isem[b]).wait()

        _issue_idx(0, 0)
        _issue_idx(1, 1)

        def _iter(u, _):
            for b in range(2):
                t = u * 2 + b
                _wait_idx(t, b)
                pltpu.async_copy(vb[b], acc.at[ib[b]], ssem[b], add=True).wait()

                @pl.when(t + 2 < n_chunks)
                def _():
                    _issue_idx(t + 2, b)
            return 0
        lax.fori_loop(0, n_chunks // 2, _iter, 0)

        plsc.subcore_barrier()
        pltpu.sync_copy(
            acc.at[pl.ds(w0, _FSLAB)],
            out.at[pl.ds(cid * NP * C + w0, _FSLAB)],
        )

    return fscat


_BM = 400


def _tc_combine_body(p_ref, o_ref):
    o_ref[...] = p_ref[0] + p_ref[1]


_tc_combine = pl.pallas_call(
    _tc_combine_body,
    grid=(N // _BM,),
    in_specs=[pl.BlockSpec((NC, _BM, C), lambda i: (0, i, 0))],
    out_specs=pl.BlockSpec((_BM, C), lambda i: (i, 0)),
    out_shape=jax.ShapeDtypeStruct((N, C), jnp.float32),
)


def _tc_matmul_body(p_ref, w_ref, b_ref, o_ref):
    x = p_ref[0] + p_ref[1]
    y = jnp.dot(x, w_ref[...], preferred_element_type=jnp.float32)
    o_ref[...] = jnp.maximum(y + b_ref[...], 0.0)


_tc_matmul = pl.pallas_call(
    _tc_matmul_body,
    grid=(N // _BM,),
    in_specs=[
        pl.BlockSpec((NC, _BM, C), lambda i: (0, i, 0)),
        pl.BlockSpec((C, C), lambda i: (0, 0)),
        pl.BlockSpec((1, C), lambda i: (0, 0)),
    ],
    out_specs=pl.BlockSpec((_BM, C), lambda i: (i, 0)),
    out_shape=jax.ShapeDtypeStruct((N, C), jnp.float32),
)


_E_PAD = _pad_edges(320000)
_NNZ_PAD = _pad_edges(128000)
_feat_scatter = _make_feat_scatter(_NNZ_PAD)
_spmm_adj = _make_spmm(_E_PAD)


def kernel(adj_indices, adj_values, feat_rows, feat_cols, feat_values, weight, bias):
    nnz = feat_values.shape[0]
    fidx = feat_rows.astype(jnp.int32) * C + feat_cols.astype(jnp.int32)
    fidx = jnp.pad(fidx, (0, _NNZ_PAD - nnz))
    fv = jnp.pad(feat_values, (0, _NNZ_PAD - nnz))
    ap = (adj_indices[0].astype(jnp.int32) << 14) | adj_indices[1].astype(jnp.int32)
    ap = jnp.pad(ap, (0, _E_PAD - 320000))
    av = jnp.pad(adj_values, (0, _E_PAD - 320000))

    pf = _feat_scatter(fidx, fv).reshape(NC, NP, C)
    base = _tc_matmul(pf, weight, bias)
    for _ in range(2):
        p = _spmm_adj(base, ap, av)
        base = _tc_combine(p)
    return base
